# baseline (speedup 1.0000x reference)
import functools

import jax
import jax.numpy as jnp
from jax import lax
from jax.experimental import pallas as pl
from jax.experimental.pallas import tpu as pltpu
from jax.experimental.pallas import tpu_sc as plsc

NC = 2
NS = 16
NW = NC * NS
CHUNK = 128


def _sc_gather(idx3, emb_table, n_chunks, D):
    per_w = n_chunks * CHUNK
    BT = NW * per_w
    mesh = plsc.VectorSubcoreMesh(core_axis_name="c", subcore_axis_name="s")

    @functools.partial(
        pl.kernel,
        mesh=mesh,
        compiler_params=pltpu.CompilerParams(use_tc_tiling_on_sc=False),
        out_type=jax.ShapeDtypeStruct((BT, D), jnp.float32),
        scratch_types=[
            pltpu.VMEM((n_chunks, CHUNK), jnp.int32),
            pltpu.VMEM((n_chunks, CHUNK, D), jnp.float32),
            pltpu.SemaphoreType.DMA,
            pltpu.SemaphoreType.DMA,
        ],
    )
    def gather_sc(idx_hbm, table_hbm, out_hbm, idx_v, rows_v, gsem, osem):
        wid = lax.axis_index("s") * NC + lax.axis_index("c")
        base = wid * per_w
        pltpu.sync_copy(idx_hbm.at[wid], idx_v)
        gathers = [
            pltpu.async_copy(table_hbm.at[idx_v.at[j]], rows_v.at[j], gsem)
            for j in range(n_chunks)
        ]
        for g in gathers:
            g.wait()
        outs = [
            pltpu.async_copy(
                rows_v.at[j], out_hbm.at[pl.ds(base + j * CHUNK, CHUNK)], osem
            )
            for j in range(n_chunks)
        ]
        for o in outs:
            o.wait()

    return gather_sc(idx3, emb_table)


def kernel(indices, emb_table, Wx, Wh, b):
    B, T = indices.shape
    V, D = emb_table.shape
    U = Wh.shape[0]
    G = 4 * U
    BT = B * T
    per_w = BT // NW
    n_chunks = per_w // CHUNK

    idx3 = indices.astype(jnp.int32).T.reshape(NW, n_chunks, CHUNK)
    S = 250880
    RB = 5120
    n_tb = S // RB

    def transpose_body(t0, t1, t2, t3, eye_ref, out_ref):
        eye = eye_ref[...]
        for a, tt in enumerate((t0, t1, t2, t3)):
            out_ref[:, 32 * a:32 * a + 32] = lax.dot_general(
                tt[...], eye, (((0,), (0,)), ((), ())),
                preferred_element_type=jnp.float32,
            )

    def _in_spec(a):
        return pl.BlockSpec((D, RB), lambda i, _a=a: (0, _a * n_tb + i))

    t128 = pl.pallas_call(
        transpose_body,
        grid=(n_tb,),
        in_specs=[_in_spec(0), _in_spec(1), _in_spec(2), _in_spec(3),
                  pl.BlockSpec((D, D), lambda i: (0, 0))],
        out_specs=pl.BlockSpec((RB, 128), lambda i: (i, 0)),
        out_shape=jax.ShapeDtypeStruct((S, 128), jnp.float32),
    )(emb_table.T, emb_table.T, emb_table.T, emb_table.T,
      jnp.eye(D, dtype=jnp.float32))
    table2 = t128.reshape(4 * S, D)
    idxr = 4 * (idx3 % S) + idx3 // S
    x_tm = _sc_gather(idxr, table2, n_chunks, D)

    col_scale = jnp.concatenate(
        [jnp.full((2 * U,), 0.5), jnp.ones((U,)), jnp.full((U,), 0.5)]
    ).astype(jnp.float32)
    Wxs = Wx * col_scale
    Whs = Wh * col_scale
    bs = (b * col_scale).reshape(1, G)

    def lstm_body(x_ref, wx_ref, wh_ref, b_ref, seq_ref, h_ref, c_ref,
                  h_scr, c_scr):
        t = pl.program_id(0)

        @pl.when(t == 0)
        def _():
            h_scr[...] = jnp.zeros((B, U), jnp.float32)
            c_scr[...] = jnp.zeros((B, U), jnp.float32)

        h = h_scr[...]
        c = c_scr[...]
        z = (
            jnp.dot(x_ref[...], wx_ref[...], preferred_element_type=jnp.float32)
            + jnp.dot(h, wh_ref[...], preferred_element_type=jnp.float32)
            + b_ref[...]
        )
        tz = jnp.tanh(z)
        i = 0.5 * tz[:, 0:U] + 0.5
        f = 0.5 * tz[:, U:2 * U] + 0.5
        g = tz[:, 2 * U:3 * U]
        o = 0.5 * tz[:, 3 * U:4 * U] + 0.5
        c = f * c + i * g
        h = o * jnp.tanh(c)
        h_scr[...] = h
        c_scr[...] = c
        seq_ref[0] = h
        h_ref[...] = h
        c_ref[...] = c

    seq_tm, h_T, c_T = pl.pallas_call(
        lstm_body,
        grid=(T,),
        in_specs=[
            pl.BlockSpec((B, D), lambda t: (t, 0)),
            pl.BlockSpec((D, G), lambda t: (0, 0)),
            pl.BlockSpec((U, G), lambda t: (0, 0)),
            pl.BlockSpec((1, G), lambda t: (0, 0)),
        ],
        out_specs=[
            pl.BlockSpec((1, B, U), lambda t: (t, 0, 0)),
            pl.BlockSpec((B, U), lambda t: (0, 0)),
            pl.BlockSpec((B, U), lambda t: (0, 0)),
        ],
        out_shape=[
            jax.ShapeDtypeStruct((T, B, U), jnp.float32),
            jax.ShapeDtypeStruct((B, U), jnp.float32),
            jax.ShapeDtypeStruct((B, U), jnp.float32),
        ],
        scratch_shapes=[
            pltpu.VMEM((B, U), jnp.float32),
            pltpu.VMEM((B, U), jnp.float32),
        ],
    )(x_tm, Wxs, Whs, bs)

    return seq_tm.transpose(1, 0, 2), h_T, c_T


# speedup vs baseline: 1.6924x; 1.6924x over previous
import functools

import jax
import jax.numpy as jnp
from jax import lax
from jax.experimental import pallas as pl
from jax.experimental.pallas import tpu as pltpu
from jax.experimental.pallas import tpu_sc as plsc

NC = 2
NS = 16
NW = NC * NS
CHUNK = 128


def _sc_gather(idx3, emb_table, n_chunks, D):
    per_w = n_chunks * CHUNK
    BT = NW * per_w
    mesh = plsc.VectorSubcoreMesh(core_axis_name="c", subcore_axis_name="s")

    @functools.partial(
        pl.kernel,
        mesh=mesh,
        compiler_params=pltpu.CompilerParams(use_tc_tiling_on_sc=False),
        out_type=jax.ShapeDtypeStruct((BT, D), jnp.float32),
        scratch_types=[
            pltpu.VMEM((n_chunks, CHUNK), jnp.int32),
            pltpu.VMEM((n_chunks, CHUNK, D), jnp.float32),
            pltpu.SemaphoreType.DMA,
            pltpu.SemaphoreType.DMA,
        ],
    )
    def gather_sc(idx_hbm, table_hbm, out_hbm, idx_v, rows_v, gsem, osem):
        wid = lax.axis_index("s") * NC + lax.axis_index("c")
        base = wid * per_w
        pltpu.sync_copy(idx_hbm.at[wid], idx_v)
        gathers = [
            pltpu.async_copy(table_hbm.at[idx_v.at[j]], rows_v.at[j], gsem)
            for j in range(n_chunks)
        ]
        for g in gathers:
            g.wait()
        outs = [
            pltpu.async_copy(
                rows_v.at[j], out_hbm.at[pl.ds(base + j * CHUNK, CHUNK)], osem
            )
            for j in range(n_chunks)
        ]
        for o in outs:
            o.wait()

    return gather_sc(idx3, emb_table)


def kernel(indices, emb_table, Wx, Wh, b):
    B, T = indices.shape
    V, D = emb_table.shape
    U = Wh.shape[0]
    G = 4 * U
    BT = B * T
    per_w = BT // NW
    n_chunks = per_w // CHUNK

    idx3 = indices.astype(jnp.int32).T.reshape(NW, n_chunks, CHUNK)
    S = 250880
    RB = 5120
    n_tb = S // RB

    def transpose_body(t0, t1, t2, t3, eye_ref, out_ref):
        tcat = jnp.concatenate(
            [t0[...], t1[...], t2[...], t3[...]], axis=0)
        out_ref[...] = lax.dot_general(
            tcat, eye_ref[...], (((0,), (0,)), ((), ())),
            preferred_element_type=jnp.float32,
        )

    def _in_spec(a):
        return pl.BlockSpec((D, RB), lambda i, _a=a: (0, _a * n_tb + i))

    t128 = pl.pallas_call(
        transpose_body,
        grid=(n_tb,),
        in_specs=[_in_spec(0), _in_spec(1), _in_spec(2), _in_spec(3),
                  pl.BlockSpec((128, 128), lambda i: (0, 0))],
        out_specs=pl.BlockSpec((RB, 128), lambda i: (i, 0)),
        out_shape=jax.ShapeDtypeStruct((S, 128), jnp.float32),
    )(emb_table.T, emb_table.T, emb_table.T, emb_table.T,
      jnp.eye(128, dtype=jnp.float32))
    table2 = t128.reshape(4 * S, D)
    idxr = 4 * (idx3 % S) + idx3 // S
    x_tm = _sc_gather(idxr, table2, n_chunks, D)

    col_scale = jnp.concatenate(
        [jnp.full((2 * U,), 0.5), jnp.ones((U,)), jnp.full((U,), 0.5)]
    ).astype(jnp.float32)
    Wxs = Wx * col_scale
    Whs = Wh * col_scale
    bs = (b * col_scale).reshape(1, G)

    def lstm_body(x_ref, wx_ref, wh_ref, b_ref, seq_ref, h_ref, c_ref,
                  h_scr, c_scr):
        t = pl.program_id(0)

        @pl.when(t == 0)
        def _():
            h_scr[...] = jnp.zeros((B, U), jnp.float32)
            c_scr[...] = jnp.zeros((B, U), jnp.float32)

        h = h_scr[...]
        c = c_scr[...]
        z = (
            jnp.dot(x_ref[...], wx_ref[...], preferred_element_type=jnp.float32)
            + jnp.dot(h, wh_ref[...], preferred_element_type=jnp.float32)
            + b_ref[...]
        )
        tz = jnp.tanh(z)
        i = 0.5 * tz[:, 0:U] + 0.5
        f = 0.5 * tz[:, U:2 * U] + 0.5
        g = tz[:, 2 * U:3 * U]
        o = 0.5 * tz[:, 3 * U:4 * U] + 0.5
        c = f * c + i * g
        h = o * jnp.tanh(c)
        h_scr[...] = h
        c_scr[...] = c
        seq_ref[0] = h
        h_ref[...] = h
        c_ref[...] = c

    seq_tm, h_T, c_T = pl.pallas_call(
        lstm_body,
        grid=(T,),
        in_specs=[
            pl.BlockSpec((B, D), lambda t: (t, 0)),
            pl.BlockSpec((D, G), lambda t: (0, 0)),
            pl.BlockSpec((U, G), lambda t: (0, 0)),
            pl.BlockSpec((1, G), lambda t: (0, 0)),
        ],
        out_specs=[
            pl.BlockSpec((1, B, U), lambda t: (t, 0, 0)),
            pl.BlockSpec((B, U), lambda t: (0, 0)),
            pl.BlockSpec((B, U), lambda t: (0, 0)),
        ],
        out_shape=[
            jax.ShapeDtypeStruct((T, B, U), jnp.float32),
            jax.ShapeDtypeStruct((B, U), jnp.float32),
            jax.ShapeDtypeStruct((B, U), jnp.float32),
        ],
        scratch_shapes=[
            pltpu.VMEM((B, U), jnp.float32),
            pltpu.VMEM((B, U), jnp.float32),
        ],
    )(x_tm, Wxs, Whs, bs)

    return seq_tm.transpose(1, 0, 2), h_T, c_T


# speedup vs baseline: 1.9401x; 1.1464x over previous
import functools

import jax
import jax.numpy as jnp
from jax import lax
from jax.experimental import pallas as pl
from jax.experimental.pallas import tpu as pltpu
from jax.experimental.pallas import tpu_sc as plsc

NC = 2
NS = 16
NW = NC * NS
CHUNK = 128


def _sc_gather(idx3, emb_table, n_chunks, D):
    per_w = n_chunks * CHUNK
    BT = NW * per_w
    mesh = plsc.VectorSubcoreMesh(core_axis_name="c", subcore_axis_name="s")

    @functools.partial(
        pl.kernel,
        mesh=mesh,
        compiler_params=pltpu.CompilerParams(use_tc_tiling_on_sc=False),
        out_type=jax.ShapeDtypeStruct((BT, 128), jnp.float32),
        scratch_types=[
            pltpu.VMEM((n_chunks, CHUNK), jnp.int32),
            pltpu.VMEM((n_chunks, CHUNK, D), jnp.float32),
            pltpu.SemaphoreType.DMA,
            pltpu.SemaphoreType.DMA,
        ],
    )
    def gather_sc(idx_hbm, table_hbm, out_hbm, idx_v, rows_v, gsem, osem):
        wid = lax.axis_index("s") * NC + lax.axis_index("c")
        base = wid * per_w
        pltpu.sync_copy(idx_hbm.at[wid], idx_v)
        gathers = [
            pltpu.async_copy(table_hbm.at[idx_v.at[j]], rows_v.at[j], gsem)
            for j in range(n_chunks)
        ]
        for g in gathers:
            g.wait()
        outs = [
            pltpu.async_copy(
                rows_v.at[j],
                out_hbm.at[pl.ds(base + j * CHUNK, CHUNK), pl.ds(0, D)],
                osem,
            )
            for j in range(n_chunks)
        ]
        for o in outs:
            o.wait()

    return gather_sc(idx3, emb_table)


def kernel(indices, emb_table, Wx, Wh, b):
    B, T = indices.shape
    V, D = emb_table.shape
    U = Wh.shape[0]
    G = 4 * U
    BT = B * T
    per_w = BT // NW
    n_chunks = per_w // CHUNK

    idx3 = indices.astype(jnp.int32).T.reshape(NW, n_chunks, CHUNK)
    S = 250880
    RB = 5120
    n_tb = S // RB

    def transpose_body(t0, t1, t2, t3, eye_ref, out_ref):
        tcat = jnp.concatenate(
            [t0[...], t1[...], t2[...], t3[...]], axis=0)
        out_ref[...] = lax.dot_general(
            tcat, eye_ref[...], (((0,), (0,)), ((), ())),
            preferred_element_type=jnp.float32,
        )

    def _in_spec(a):
        return pl.BlockSpec((D, RB), lambda i, _a=a: (0, _a * n_tb + i))

    t128 = pl.pallas_call(
        transpose_body,
        grid=(n_tb,),
        in_specs=[_in_spec(0), _in_spec(1), _in_spec(2), _in_spec(3),
                  pl.BlockSpec((128, 128), lambda i: (0, 0))],
        out_specs=pl.BlockSpec((RB, 128), lambda i: (i, 0)),
        out_shape=jax.ShapeDtypeStruct((S, 128), jnp.float32),
    )(emb_table.T, emb_table.T, emb_table.T, emb_table.T,
      jnp.eye(128, dtype=jnp.float32))
    table2 = t128.reshape(4 * S, D)
    idxr = 4 * (idx3 % S) + idx3 // S
    x_tm = _sc_gather(idxr, table2, n_chunks, D)

    col_scale = jnp.concatenate(
        [jnp.full((2 * U,), 0.5), jnp.ones((U,)), jnp.full((U,), 0.5)]
    ).astype(jnp.float32)
    Wxs = Wx * col_scale
    Whs = Wh * col_scale
    bs = (b * col_scale).reshape(1, G)

    def lstm_body(x_ref, wx_ref, wh_ref, b_ref, seq_ref, h_ref, c_ref,
                  h_scr, c_scr):
        t = pl.program_id(0)

        @pl.when(t == 0)
        def _():
            h_scr[...] = jnp.zeros((B, U), jnp.float32)
            c_scr[...] = jnp.zeros((B, U), jnp.float32)

        h = h_scr[...]
        c = c_scr[...]
        z = (
            jnp.dot(x_ref[:, 0:D], wx_ref[...], preferred_element_type=jnp.float32)
            + jnp.dot(h, wh_ref[...], preferred_element_type=jnp.float32)
            + b_ref[...]
        )
        tz = jnp.tanh(z)
        i = 0.5 * tz[:, 0:U] + 0.5
        f = 0.5 * tz[:, U:2 * U] + 0.5
        g = tz[:, 2 * U:3 * U]
        o = 0.5 * tz[:, 3 * U:4 * U] + 0.5
        c = f * c + i * g
        h = o * jnp.tanh(c)
        h_scr[...] = h
        c_scr[...] = c
        seq_ref[0] = h
        h_ref[...] = h
        c_ref[...] = c

    seq_tm, h_T, c_T = pl.pallas_call(
        lstm_body,
        grid=(T,),
        in_specs=[
            pl.BlockSpec((B, 128), lambda t: (t, 0)),
            pl.BlockSpec((D, G), lambda t: (0, 0)),
            pl.BlockSpec((U, G), lambda t: (0, 0)),
            pl.BlockSpec((1, G), lambda t: (0, 0)),
        ],
        out_specs=[
            pl.BlockSpec((1, B, U), lambda t: (t, 0, 0)),
            pl.BlockSpec((B, U), lambda t: (0, 0)),
            pl.BlockSpec((B, U), lambda t: (0, 0)),
        ],
        out_shape=[
            jax.ShapeDtypeStruct((T, B, U), jnp.float32),
            jax.ShapeDtypeStruct((B, U), jnp.float32),
            jax.ShapeDtypeStruct((B, U), jnp.float32),
        ],
        scratch_shapes=[
            pltpu.VMEM((B, U), jnp.float32),
            pltpu.VMEM((B, U), jnp.float32),
        ],
    )(x_tm, Wxs, Whs, bs)

    return seq_tm.transpose(1, 0, 2), h_T, c_T
